# Initial kernel scaffold; baseline (speedup 1.0000x reference)
import jax
import jax.numpy as jnp
from jax.experimental import pallas as pl


def kernel(x, edge_index, batch, g0_W1, g0_b1, g0_W2, g0_b2, g1_W1, g1_b1, g1_W2, g1_b2, g2_W1, g2_b1, g2_W2, g2_b2, set_W, set_fc_W, set_fc_b, mlp_W1, mlp_b1, mlp_W2, mlp_b2):
    raise NotImplementedError("write your pallas kernel here")


# speedup vs baseline: 6.9741x; 6.9741x over previous
import functools

import jax
import jax.numpy as jnp
from jax import lax
from jax.experimental import pallas as pl
from jax.experimental.pallas import tpu as pltpu
from jax.experimental.pallas import tpu_sc as plsc

N_NODES = 10000
D_IN = 128
D_H = 32
N_EDGES = 320000
B_GRAPHS = 64
N_HS = 32
N_EL = 16

NC = 2
NS = 16
EDGES_PER_TILE = N_EDGES // (NC * NS)
N_PAD = 10240
ROWS_PER_TILE = N_PAD // NS


CH = 80
ZR = 64


@functools.lru_cache(maxsize=None)
def _make_edge_segsum(d, rows_tile, rows_core, n_rows):
    nz = ROWS_PER_TILE // ZR

    @functools.partial(
        pl.kernel,
        out_type=jax.ShapeDtypeStruct((NC * N_PAD, d), jnp.float32),
        mesh=plsc.VectorSubcoreMesh(core_axis_name="c", subcore_axis_name="s"),
        compiler_params=pltpu.CompilerParams(use_tc_tiling_on_sc=False),
        scratch_types=[
            pltpu.VMEM((CH,), jnp.int32),
            pltpu.VMEM((CH,), jnp.int32),
            pltpu.VMEM((CH, d), jnp.float32),
            pltpu.VMEM((ZR, d), jnp.float32),
            pltpu.VMEM_SHARED((N_PAD, d), jnp.float32),
            pltpu.SemaphoreType.DMA,
        ],
    )
    def _sc_body(h_hbm, src_hbm, dst_hbm, out_hbm,
                 src_v, dst_v, rows_v, zbuf, acc_sh, sem):
        c = lax.axis_index("c")
        s = lax.axis_index("s")

        zvec = jnp.zeros((16,), jnp.float32)

        def _zero_row(i, _):
            for v in range(d // 16):
                zbuf[i, pl.ds(v * 16, 16)] = zvec
            return 0

        lax.fori_loop(0, ZR, _zero_row, 0)
        for z in range(nz):
            pltpu.sync_copy(
                zbuf, acc_sh.at[pl.ds(s * ROWS_PER_TILE + z * ZR, ZR)])
        plsc.subcore_barrier()

        row0 = c * rows_core + s * rows_tile

        def _chunk(j, _):
            pltpu.sync_copy(src_hbm.at[row0 + j], src_v)
            pltpu.sync_copy(dst_hbm.at[row0 + j], dst_v)
            pltpu.async_copy(h_hbm.at[src_v], rows_v, sem).wait()
            pltpu.sync_copy(rows_v, acc_sh.at[dst_v], add=True)
            return 0

        lax.fori_loop(0, rows_tile, _chunk, 0)
        plsc.subcore_barrier()

        r0 = s * ROWS_PER_TILE
        pltpu.sync_copy(acc_sh.at[pl.ds(r0, ROWS_PER_TILE)],
                        out_hbm.at[pl.ds(c * N_PAD + r0, ROWS_PER_TILE)])

    return _sc_body


def _edge_segsum(h, src2d, dst2d):
    return _make_edge_segsum(D_H, EDGES_PER_TILE // CH,
                             N_EDGES // 2 // CH, N_EDGES // CH)(
                                 h, src2d, dst2d)


def _edge_segsum_wide(x2, src2d, dst2d):
    return _make_edge_segsum(D_IN // 2, N_EDGES // NS // CH,
                             N_EDGES // CH, 2 * N_EDGES // CH)(
                                 x2, src2d, dst2d)


def _layer0_body(h_ref, part_ref, w1_ref, b1_ref, w2_ref, b2_ref, o_ref):
    agg = jnp.concatenate([part_ref[0:N_NODES, :],
                           part_ref[N_PAD:N_PAD + N_NODES, :]], axis=1)
    z = h_ref[...] + agg
    z = jnp.maximum(jnp.dot(z, w1_ref[...],
                            preferred_element_type=jnp.float32) + b1_ref[...],
                    0.0)
    z = jnp.dot(z, w2_ref[...], preferred_element_type=jnp.float32) + b2_ref[...]
    o_ref[...] = jnp.maximum(z, 0.0)


def _layer_body(h_ref, part_ref, w1_ref, b1_ref, w2_ref, b2_ref, o_ref):
    agg = part_ref[0:N_NODES, :] + part_ref[N_PAD:N_PAD + N_NODES, :]
    z = h_ref[...] + agg
    z = jnp.maximum(jnp.dot(z, w1_ref[...],
                            preferred_element_type=jnp.float32) + b1_ref[...],
                    0.0)
    z = jnp.dot(z, w2_ref[...], preferred_element_type=jnp.float32) + b2_ref[...]
    o_ref[...] = jnp.maximum(z, 0.0)


def _final_body(h_ref, wset_ref, batch_ref, fcw_ref, fcb_ref,
                mw1_ref, mb1_ref, mw2_ref, mb2_ref, o_ref):
    h = h_ref[...]
    m = None
    for l in range(N_EL):
        t = jnp.maximum(jnp.dot(h, wset_ref[l],
                                preferred_element_type=jnp.float32), 0.0)
        m = t if m is None else jnp.maximum(m, t)

    gid = lax.broadcasted_iota(jnp.int32, (B_GRAPHS, N_NODES), 0)
    onehot = (batch_ref[...] == gid).astype(jnp.float32)
    pooled = jnp.dot(onehot, m, precision=lax.Precision.HIGHEST,
                     preferred_element_type=jnp.float32)

    t = jnp.maximum(jnp.dot(pooled, fcw_ref[...],
                            preferred_element_type=jnp.float32) + fcb_ref[...],
                    0.0)
    o = jnp.maximum(jnp.dot(t, mw1_ref[...],
                            preferred_element_type=jnp.float32) + mb1_ref[...],
                    0.0)
    o = jnp.dot(o, mw2_ref[...], preferred_element_type=jnp.float32) + mb2_ref[...]
    mx = jnp.max(o, axis=1, keepdims=True)
    shifted = o - mx
    o_ref[...] = shifted - jnp.log(jnp.sum(jnp.exp(shifted), axis=1,
                                           keepdims=True))


def _tc_call(body, out_shape):
    return pl.pallas_call(body, out_shape=out_shape)


def kernel(x, edge_index, batch, g0_W1, g0_b1, g0_W2, g0_b2, g1_W1, g1_b1,
           g1_W2, g1_b2, g2_W1, g2_b1, g2_W2, g2_b2, set_W, set_fc_W,
           set_fc_b, mlp_W1, mlp_b1, mlp_W2, mlp_b2):
    src = edge_index[0]
    dst = edge_index[1]
    f32 = jnp.float32

    src2d = src.reshape(N_EDGES // CH, CH)
    dst2d = dst.reshape(N_EDGES // CH, CH)
    x2 = jnp.concatenate([x[:, :D_IN // 2], x[:, D_IN // 2:]], axis=0)
    src2 = jnp.concatenate([src2d, src2d + N_NODES], axis=0)
    dst2 = jnp.concatenate([dst2d, dst2d], axis=0)
    part = _edge_segsum_wide(x2, src2, dst2)
    h = _tc_call(_layer0_body, jax.ShapeDtypeStruct((N_NODES, D_H), f32))(
        x, part, g0_W1, g0_b1.reshape(1, D_H), g0_W2, g0_b2.reshape(1, D_H))
    for (W1, b1, W2, b2) in ((g1_W1, g1_b1, g1_W2, g1_b2),
                             (g2_W1, g2_b1, g2_W2, g2_b2)):
        part = _edge_segsum(h, src2d, dst2d)
        h = _tc_call(_layer_body, jax.ShapeDtypeStruct((N_NODES, D_H), f32))(
            h, part, W1, b1.reshape(1, D_H), W2, b2.reshape(1, D_H))

    wset = set_W.reshape(D_H, N_HS, N_EL).transpose(2, 0, 1)
    out = _tc_call(_final_body, jax.ShapeDtypeStruct((B_GRAPHS, 2), f32))(
        h, wset, batch.reshape(1, N_NODES).astype(jnp.int32),
        set_fc_W, set_fc_b.reshape(1, D_H),
        mlp_W1, mlp_b1.reshape(1, -1), mlp_W2, mlp_b2.reshape(1, -1))
    return out
